# Initial kernel scaffold; baseline (speedup 1.0000x reference)
import jax
import jax.numpy as jnp
from jax.experimental import pallas as pl


def kernel(x, table):
    raise NotImplementedError("write your pallas kernel here")


# speedup vs baseline: 4.9462x; 4.9462x over previous
import jax
import jax.numpy as jnp
from jax import lax
from jax.experimental import pallas as pl
from jax.experimental.pallas import tpu as pltpu
from jax.experimental.pallas import tpu_sc as plsc

_NC = 2
_NS = 16
_NW = _NC * _NS

_CH = 2048


def _gather_body(idx_hbm, table_hbm, out_hbm, idx_v, rows_v, sem):
    wid = lax.axis_index("s") * _NC + lax.axis_index("c")
    n = idx_hbm.shape[0]
    b_per_w = n // _NW
    base = wid * b_per_w
    nch = b_per_w // _CH

    def body(c, carry):
        off = base + c * _CH
        pltpu.sync_copy(idx_hbm.at[pl.ds(off, _CH)], idx_v)
        pltpu.async_copy(table_hbm.at[idx_v], rows_v, sem).wait()
        pltpu.sync_copy(rows_v, out_hbm.at[pl.ds(off, _CH)])
        return carry

    lax.fori_loop(0, nch, body, 0)


def kernel(x, table):
    b, h = x.shape
    _, d = table.shape
    n = b * h
    flat = x.reshape(n)

    mesh = plsc.VectorSubcoreMesh(core_axis_name="c", subcore_axis_name="s")
    f = pl.kernel(
        _gather_body,
        mesh=mesh,
        out_type=jax.ShapeDtypeStruct((n, d), jnp.float32),
        scratch_types=[
            pltpu.VMEM((_CH,), jnp.int32),
            pltpu.VMEM((_CH, d), jnp.float32),
            pltpu.SemaphoreType.DMA,
        ],
        compiler_params=pltpu.CompilerParams(use_tc_tiling_on_sc=False),
    )
    out = f(flat, table)
    return out.reshape(b, h, d)


# speedup vs baseline: 5.0365x; 1.0183x over previous
import jax
import jax.numpy as jnp
from jax import lax
from jax.experimental import pallas as pl
from jax.experimental.pallas import tpu as pltpu
from jax.experimental.pallas import tpu_sc as plsc

_NC = 2
_NS = 16
_NW = _NC * _NS

_CH = 1600


def _gather_body(idx_hbm, table_hbm, out_hbm,
                 idx0, idx1, rows0, rows1,
                 si0, si1, sg0, sg1, so0, so1):
    wid = lax.axis_index("s") * _NC + lax.axis_index("c")
    n = idx_hbm.shape[0]
    b_per_w = n // _NW
    base = wid * b_per_w
    npairs = b_per_w // (2 * _CH)

    def idx_copy(g, buf, sem):
        return pltpu.make_async_copy(
            idx_hbm.at[pl.ds(base + g * _CH, _CH)], buf, sem)

    def gather_copy(ibuf, rbuf, sem):
        return pltpu.make_async_copy(table_hbm.at[ibuf], rbuf, sem)

    def out_copy(g, rbuf, sem):
        return pltpu.make_async_copy(
            rbuf, out_hbm.at[pl.ds(base + g * _CH, _CH)], sem)

    idx_copy(0, idx0, si0).start()
    idx_copy(1, idx1, si1).start()
    idx_copy(0, idx0, si0).wait()
    gather_copy(idx0, rows0, sg0).start()

    def pair(p, carry):
        g0 = 2 * p
        g1 = g0 + 1

        idx_copy(g1, idx1, si1).wait()

        @pl.when(p > 0)
        def _():
            out_copy(g1 - 2, rows1, so1).wait()

        gather_copy(idx1, rows1, sg1).start()

        gather_copy(idx0, rows0, sg0).wait()
        out_copy(g0, rows0, so0).start()

        gather_copy(idx1, rows1, sg1).wait()
        out_copy(g1, rows1, so1).start()

        @pl.when(p < npairs - 1)
        def _():
            idx_copy(g0 + 2, idx0, si0).start()
            idx_copy(g1 + 2, idx1, si1).start()
            idx_copy(g0 + 2, idx0, si0).wait()
            out_copy(g0, rows0, so0).wait()
            gather_copy(idx0, rows0, sg0).start()

        return carry

    lax.fori_loop(0, npairs, pair, 0)

    out_copy(2 * npairs - 2, rows0, so0).wait()
    out_copy(2 * npairs - 1, rows1, so1).wait()


def kernel(x, table):
    b, h = x.shape
    _, d = table.shape
    n = b * h
    flat = x.reshape(n)

    mesh = plsc.VectorSubcoreMesh(core_axis_name="c", subcore_axis_name="s")
    f = pl.kernel(
        _gather_body,
        mesh=mesh,
        out_type=jax.ShapeDtypeStruct((n, d), jnp.float32),
        scratch_types=[
            pltpu.VMEM((_CH,), jnp.int32),
            pltpu.VMEM((_CH,), jnp.int32),
            pltpu.VMEM((_CH, d), jnp.float32),
            pltpu.VMEM((_CH, d), jnp.float32),
            pltpu.SemaphoreType.DMA,
            pltpu.SemaphoreType.DMA,
            pltpu.SemaphoreType.DMA,
            pltpu.SemaphoreType.DMA,
            pltpu.SemaphoreType.DMA,
            pltpu.SemaphoreType.DMA,
        ],
        compiler_params=pltpu.CompilerParams(use_tc_tiling_on_sc=False),
    )
    out = f(flat, table)
    return out.reshape(b, h, d)


# speedup vs baseline: 5.0523x; 1.0031x over previous
import jax
import jax.numpy as jnp
from jax import lax
from jax.experimental import pallas as pl
from jax.experimental.pallas import tpu as pltpu
from jax.experimental.pallas import tpu_sc as plsc

_NC = 2
_NS = 16
_NW = _NC * _NS

_NBUF = 4
_CH = 800


def _gather_body(idx_hbm, table_hbm, out_hbm, *refs):
    idx_bufs = refs[0:_NBUF]
    row_bufs = refs[_NBUF:2 * _NBUF]
    si = refs[2 * _NBUF:3 * _NBUF]
    sg = refs[3 * _NBUF:4 * _NBUF]
    so = refs[4 * _NBUF:5 * _NBUF]

    wid = lax.axis_index("s") * _NC + lax.axis_index("c")
    n = idx_hbm.shape[0]
    b_per_w = n // _NW
    base = wid * b_per_w
    nch = b_per_w // _CH

    def idx_copy(c, b):
        return pltpu.make_async_copy(
            idx_hbm.at[pl.ds(base + c * _CH, _CH)], idx_bufs[b], si[b])

    def gather_copy(b):
        return pltpu.make_async_copy(
            table_hbm.at[idx_bufs[b]], row_bufs[b], sg[b])

    def out_copy(c, b):
        return pltpu.make_async_copy(
            row_bufs[b], out_hbm.at[pl.ds(base + c * _CH, _CH)], so[b])

    for b in range(_NBUF):
        idx_copy(b, b).start()
    for b in range(_NBUF):
        idx_copy(b, b).wait()
        gather_copy(b).start()

    def outer(o, carry):
        for b in range(_NBUF):
            c = o * _NBUF + b
            gather_copy(b).wait()
            out_copy(c, b).start()

            @pl.when(c + _NBUF < nch)
            def _():
                idx_copy(c + _NBUF, b).start()

            pb = (b - 1) % _NBUF
            pc = c - 1

            @pl.when(jnp.logical_and(pc >= 0, pc + _NBUF < nch))
            def _():
                out_copy(pc, pb).wait()
                idx_copy(pc + _NBUF, pb).wait()
                gather_copy(pb).start()

        return carry

    lax.fori_loop(0, nch // _NBUF, outer, 0)

    for b in range(_NBUF):
        c = nch - _NBUF + b
        out_copy(c, b).wait()


def kernel(x, table):
    b, h = x.shape
    _, d = table.shape
    n = b * h
    flat = x.reshape(n)

    mesh = plsc.VectorSubcoreMesh(core_axis_name="c", subcore_axis_name="s")
    scratch = (
        [pltpu.VMEM((_CH,), jnp.int32) for _ in range(_NBUF)]
        + [pltpu.VMEM((_CH, d), jnp.float32) for _ in range(_NBUF)]
        + [pltpu.SemaphoreType.DMA for _ in range(3 * _NBUF)]
    )
    f = pl.kernel(
        _gather_body,
        mesh=mesh,
        out_type=jax.ShapeDtypeStruct((n, d), jnp.float32),
        scratch_types=scratch,
        compiler_params=pltpu.CompilerParams(use_tc_tiling_on_sc=False),
    )
    out = f(flat, table)
    return out.reshape(b, h, d)
